# Initial kernel scaffold; baseline (speedup 1.0000x reference)
import jax
import jax.numpy as jnp
from jax.experimental import pallas as pl


def kernel(input_ids, embedding):
    raise NotImplementedError("write your pallas kernel here")


# speedup vs baseline: 1.6239x; 1.6239x over previous
import functools

import jax
import jax.numpy as jnp
from jax import lax
from jax.experimental import pallas as pl
from jax.experimental.pallas import tpu as pltpu
from jax.experimental.pallas import tpu_sc as plsc

VOCAB = 100000
HIDDEN = 1024
BATCH = 4
SEQ = 8192
TOTAL = BATCH * SEQ

NUM_CORES = 2
NUM_SUBCORES = 16
NUM_WORKERS = NUM_CORES * NUM_SUBCORES
PER_WORKER = TOTAL // NUM_WORKERS

CHUNK = 64
NCHUNK = PER_WORKER // CHUNK


_mesh = plsc.VectorSubcoreMesh(core_axis_name="c", subcore_axis_name="s")


@functools.partial(
    pl.kernel,
    out_type=jax.ShapeDtypeStruct((TOTAL, HIDDEN), jnp.float32),
    mesh=_mesh,
    scratch_types=[
        pltpu.VMEM((PER_WORKER,), jnp.int32),
        pltpu.VMEM((CHUNK, HIDDEN), jnp.float32),
        pltpu.SemaphoreType.DMA,
    ],
)
def _embed_gather(table_hbm, idx_hbm, out_hbm, idx_v, rows_v, sem):
    wid = lax.axis_index("s") * NUM_CORES + lax.axis_index("c")
    base = wid * PER_WORKER
    pltpu.sync_copy(idx_hbm.at[pl.ds(base, PER_WORKER)], idx_v)

    @pl.loop(0, NCHUNK)
    def _step(i):
        off = pl.multiple_of(i * CHUNK, 8)
        pltpu.async_copy(
            table_hbm.at[idx_v.at[pl.ds(off, CHUNK)]], rows_v, sem
        ).wait()
        pltpu.sync_copy(rows_v, out_hbm.at[pl.ds(base + off, CHUNK)])


def kernel(input_ids, embedding):
    idx = input_ids.reshape(TOTAL).astype(jnp.int32)
    out = _embed_gather(embedding, idx)
    return out.reshape(BATCH, SEQ, HIDDEN)


# speedup vs baseline: 1.7753x; 1.0932x over previous
import functools

import jax
import jax.numpy as jnp
from jax import lax
from jax.experimental import pallas as pl
from jax.experimental.pallas import tpu as pltpu
from jax.experimental.pallas import tpu_sc as plsc

VOCAB = 100000
HIDDEN = 1024
BATCH = 4
SEQ = 8192
TOTAL = BATCH * SEQ

NUM_CORES = 2
NUM_SUBCORES = 16
NUM_WORKERS = NUM_CORES * NUM_SUBCORES
PER_WORKER = TOTAL // NUM_WORKERS

CHUNK = 16
NCHUNK = PER_WORKER // CHUNK
NBUF = 4


_mesh = plsc.VectorSubcoreMesh(core_axis_name="c", subcore_axis_name="s")


@functools.partial(
    pl.kernel,
    out_type=jax.ShapeDtypeStruct((TOTAL, HIDDEN), jnp.float32),
    mesh=_mesh,
    scratch_types=[
        pltpu.VMEM((PER_WORKER,), jnp.int32),
        *[pltpu.VMEM((CHUNK, HIDDEN), jnp.float32) for _ in range(NBUF)],
        *[pltpu.SemaphoreType.DMA for _ in range(NBUF)],
    ],
)
def _embed_gather(table_hbm, idx_hbm, out_hbm, idx_v, *bufs_and_sems):
    rows_v = bufs_and_sems[:NBUF]
    gsem = bufs_and_sems[NBUF:]
    wid = lax.axis_index("s") * NUM_CORES + lax.axis_index("c")
    base = wid * PER_WORKER
    pltpu.sync_copy(idx_hbm.at[pl.ds(base, PER_WORKER)], idx_v)

    def start_gather(chunk, b):
        off = pl.multiple_of(chunk * CHUNK, 8)
        pltpu.make_async_copy(
            table_hbm.at[idx_v.at[pl.ds(off, CHUNK)]],
            rows_v[b],
            gsem[b],
        ).start()

    def drain(chunk, b):
        off = pl.multiple_of(chunk * CHUNK, 8)
        pltpu.make_async_copy(
            table_hbm.at[idx_v.at[pl.ds(off, CHUNK)]],
            rows_v[b],
            gsem[b],
        ).wait()
        pltpu.sync_copy(rows_v[b], out_hbm.at[pl.ds(base + off, CHUNK)])

    for b in range(NBUF):
        start_gather(b, b)

    @pl.loop(0, NCHUNK - NBUF, step=NBUF)
    def _round(i):
        for b in range(NBUF):
            drain(i + b, b)
            start_gather(i + b + NBUF, b)

    for b in range(NBUF):
        drain(NCHUNK - NBUF + b, b)


def kernel(input_ids, embedding):
    idx = input_ids.reshape(TOTAL).astype(jnp.int32)
    out = _embed_gather(embedding, idx)
    return out.reshape(BATCH, SEQ, HIDDEN)


# speedup vs baseline: 1.7795x; 1.0024x over previous
import functools

import jax
import jax.numpy as jnp
from jax import lax
from jax.experimental import pallas as pl
from jax.experimental.pallas import tpu as pltpu
from jax.experimental.pallas import tpu_sc as plsc

VOCAB = 100000
HIDDEN = 1024
BATCH = 4
SEQ = 8192
TOTAL = BATCH * SEQ

NUM_CORES = 2
NUM_SUBCORES = 16
NUM_WORKERS = NUM_CORES * NUM_SUBCORES
PER_WORKER = TOTAL // NUM_WORKERS

CHUNK = 16
NCHUNK = PER_WORKER // CHUNK
NBUF = 4


_mesh = plsc.VectorSubcoreMesh(core_axis_name="c", subcore_axis_name="s")


@functools.partial(
    pl.kernel,
    out_type=jax.ShapeDtypeStruct((TOTAL, HIDDEN), jnp.float32),
    mesh=_mesh,
    scratch_types=[
        pltpu.VMEM((PER_WORKER,), jnp.int32),
        *[pltpu.VMEM((CHUNK, HIDDEN), jnp.float32) for _ in range(NBUF)],
        *[pltpu.SemaphoreType.DMA for _ in range(2 * NBUF)],
    ],
)
def _embed_gather(table_hbm, idx_hbm, out_hbm, idx_v, *bufs_and_sems):
    rows_v = bufs_and_sems[:NBUF]
    gsem = bufs_and_sems[NBUF : 2 * NBUF]
    ssem = bufs_and_sems[2 * NBUF :]
    wid = lax.axis_index("s") * NUM_CORES + lax.axis_index("c")
    base = wid * PER_WORKER
    pltpu.sync_copy(idx_hbm.at[pl.ds(base, PER_WORKER)], idx_v)

    def gather_desc(chunk, b):
        off = pl.multiple_of(chunk * CHUNK, 8)
        return pltpu.make_async_copy(
            table_hbm.at[idx_v.at[pl.ds(off, CHUNK)]], rows_v[b], gsem[b]
        )

    def scatter_desc(chunk, b):
        off = pl.multiple_of(chunk * CHUNK, 8)
        return pltpu.make_async_copy(
            rows_v[b], out_hbm.at[pl.ds(base + off, CHUNK)], ssem[b]
        )

    for b in range(NBUF):
        gather_desc(b, b).start()

    gather_desc(0, 0).wait()
    scatter_desc(0, 0).start()

    @pl.loop(0, NCHUNK - NBUF, step=NBUF)
    def _round(i):
        for j in range(NBUF):
            r = i + j + 1
            b = (j + 1) % NBUF
            bp = j
            scatter_desc(r - 1, bp).wait()
            gather_desc(r + NBUF - 1, bp).start()
            gather_desc(r, b).wait()
            scatter_desc(r, b).start()

    for r in range(NCHUNK - NBUF + 1, NCHUNK):
        b = r % NBUF
        gather_desc(r, b).wait()
        scatter_desc(r, b).start()

    for r in range(NCHUNK - NBUF, NCHUNK):
        scatter_desc(r, r % NBUF).wait()


def kernel(input_ids, embedding):
    idx = input_ids.reshape(TOTAL).astype(jnp.int32)
    out = _embed_gather(embedding, idx)
    return out.reshape(BATCH, SEQ, HIDDEN)


# speedup vs baseline: 1.7853x; 1.0032x over previous
import functools

import jax
import jax.numpy as jnp
from jax import lax
from jax.experimental import pallas as pl
from jax.experimental.pallas import tpu as pltpu
from jax.experimental.pallas import tpu_sc as plsc

VOCAB = 100000
HIDDEN = 1024
BATCH = 4
SEQ = 8192
TOTAL = BATCH * SEQ

NUM_CORES = 2
NUM_SUBCORES = 16
NUM_WORKERS = NUM_CORES * NUM_SUBCORES
PER_WORKER = TOTAL // NUM_WORKERS

CHUNK = 32
NCHUNK = PER_WORKER // CHUNK
NBUF = 3


_mesh = plsc.VectorSubcoreMesh(core_axis_name="c", subcore_axis_name="s")


@functools.partial(
    pl.kernel,
    out_type=jax.ShapeDtypeStruct((TOTAL, HIDDEN), jnp.float32),
    mesh=_mesh,
    scratch_types=[
        pltpu.VMEM((PER_WORKER,), jnp.int32),
        *[pltpu.VMEM((CHUNK, HIDDEN), jnp.float32) for _ in range(NBUF)],
        *[pltpu.SemaphoreType.DMA for _ in range(2 * NBUF)],
    ],
)
def _embed_gather(table_hbm, idx_hbm, out_hbm, idx_v, *bufs_and_sems):
    rows_v = bufs_and_sems[:NBUF]
    gsem = bufs_and_sems[NBUF : 2 * NBUF]
    ssem = bufs_and_sems[2 * NBUF :]
    wid = lax.axis_index("s") * NUM_CORES + lax.axis_index("c")
    base = wid * PER_WORKER
    pltpu.sync_copy(idx_hbm.at[pl.ds(base, PER_WORKER)], idx_v)

    def gather_desc(chunk, b):
        off = pl.multiple_of(chunk * CHUNK, 8)
        return pltpu.make_async_copy(
            table_hbm.at[idx_v.at[pl.ds(off, CHUNK)]], rows_v[b], gsem[b]
        )

    def scatter_desc(chunk, b):
        off = pl.multiple_of(chunk * CHUNK, 8)
        return pltpu.make_async_copy(
            rows_v[b], out_hbm.at[pl.ds(base + off, CHUNK)], ssem[b]
        )

    for b in range(NBUF):
        gather_desc(b, b).start()

    gather_desc(0, 0).wait()
    scatter_desc(0, 0).start()

    main = ((NCHUNK - NBUF) // NBUF) * NBUF

    @pl.loop(0, main, step=NBUF)
    def _round(i):
        for j in range(NBUF):
            r = i + j + 1
            b = (j + 1) % NBUF
            bp = j
            scatter_desc(r - 1, bp).wait()
            gather_desc(r + NBUF - 1, bp).start()
            gather_desc(r, b).wait()
            scatter_desc(r, b).start()

    for r in range(main + 1, NCHUNK - NBUF + 1):
        b = r % NBUF
        bp = (r - 1) % NBUF
        scatter_desc(r - 1, bp).wait()
        gather_desc(r + NBUF - 1, bp).start()
        gather_desc(r, b).wait()
        scatter_desc(r, b).start()

    for r in range(NCHUNK - NBUF + 1, NCHUNK):
        b = r % NBUF
        gather_desc(r, b).wait()
        scatter_desc(r, b).start()

    for r in range(NCHUNK - NBUF, NCHUNK):
        scatter_desc(r, r % NBUF).wait()


def kernel(input_ids, embedding):
    idx = input_ids.reshape(TOTAL).astype(jnp.int32)
    out = _embed_gather(embedding, idx)
    return out.reshape(BATCH, SEQ, HIDDEN)
